# baseline (speedup 1.0000x reference)
import functools
import math

import jax
import jax.numpy as jnp
from jax.experimental import pallas as pl
from jax.experimental.pallas import tpu as pltpu

N = 4096
DEPTH = 3
RATIO = 0.5


def _mm_body(a_ref, b_ref, o_ref, acc_ref, *, nk):
    @pl.when(pl.program_id(2) == 0)
    def _():
        acc_ref[...] = jnp.zeros_like(acc_ref)

    acc_ref[...] += jnp.dot(a_ref[...], b_ref[...],
                            preferred_element_type=jnp.float32)

    @pl.when(pl.program_id(2) == nk - 1)
    def _():
        o_ref[...] = acc_ref[...]


def _mm(a, b, bm=512, bn=512, bk=512):
    m, k = a.shape
    k2, n = b.shape
    assert k == k2
    bm = min(bm, m)
    bn = min(bn, n)
    bk = min(bk, k)
    grid = (m // bm, n // bn, k // bk)
    return pl.pallas_call(
        functools.partial(_mm_body, nk=grid[2]),
        out_shape=jax.ShapeDtypeStruct((m, n), jnp.float32),
        grid=grid,
        in_specs=[
            pl.BlockSpec((bm, bk), lambda i, j, h: (i, h)),
            pl.BlockSpec((bk, bn), lambda i, j, h: (h, j)),
        ],
        out_specs=pl.BlockSpec((bm, bn), lambda i, j, h: (i, j)),
        scratch_shapes=[pltpu.VMEM((bm, bn), jnp.float32)],
    )(a, b)


def _gcn_norm(A, fill=2.0):
    diag = jnp.diagonal(A)
    A = A + jnp.diag(jnp.where(diag == 0, fill, 0.0))
    deg = A.sum(axis=1)
    dinv = jnp.where(deg > 0, 1.0 / jnp.sqrt(deg), 0.0)
    return dinv[:, None] * A * dinv[None, :]


def _gcn_conv(x, A, W, b):
    return _mm(_gcn_norm(A), _mm(x, W)) + b


def _augment(A):
    n = A.shape[0]
    eye = jnp.eye(n, dtype=A.dtype)
    A = A * (1.0 - eye) + eye
    A2 = _mm(A, A)
    return A2 * (1.0 - eye)


def _topk_pool(x, A, p, ratio):
    score = (x @ p) / jnp.linalg.norm(p)
    k = int(math.ceil(ratio * x.shape[0]))
    vals, perm = jax.lax.top_k(score, k)
    xp = x[perm] * jnp.tanh(vals)[:, None]
    Ap = A[perm][:, perm]
    return xp, Ap, perm


def kernel(x, edge_index, W_down0, b_down0, W_down1, b_down1, W_down2,
           b_down2, W_down3, b_down3, p_pool1, p_pool2, p_pool3,
           W_up0, b_up0, W_up1, b_up1, W_up2, b_up2):
    n = x.shape[0]
    A = jnp.zeros((n, n), jnp.float32).at[edge_index[1], edge_index[0]].add(
        jnp.ones((edge_index.shape[1],), jnp.float32))
    x = jax.nn.relu(_gcn_conv(x, A, W_down0, b_down0))
    xs = [x]
    As = [A]
    perms = []
    Wd = [(W_down1, b_down1), (W_down2, b_down2), (W_down3, b_down3)]
    ps = [p_pool1, p_pool2, p_pool3]
    for i in range(DEPTH):
        A = _augment(A)
        x, A, perm = _topk_pool(x, A, ps[i], RATIO)
        x = jax.nn.relu(_gcn_conv(x, A, Wd[i][0], Wd[i][1]))
        if i < DEPTH - 1:
            xs.append(x)
            As.append(A)
        perms.append(perm)
    Wu = [(W_up0, b_up0), (W_up1, b_up1), (W_up2, b_up2)]
    for i in range(DEPTH):
        j = DEPTH - 1 - i
        res = xs[j]
        A = As[j]
        perm = perms[j]
        up = jnp.zeros_like(res).at[perm].set(x)
        xcat = jnp.concatenate([res, up], axis=-1)
        x = _gcn_conv(xcat, A, Wu[i][0], Wu[i][1])
        if i < DEPTH - 1:
            x = jax.nn.relu(x)
    return x


# speedup vs baseline: 1.1379x; 1.1379x over previous
import functools
import math

import jax
import jax.numpy as jnp
from jax.experimental import pallas as pl
from jax.experimental.pallas import tpu as pltpu

DEPTH = 3
RATIO = 0.5
BLK = 512


def _mm_body(a_ref, b_ref, o_ref, acc_ref, *, nk):
    @pl.when(pl.program_id(2) == 0)
    def _():
        acc_ref[...] = jnp.zeros_like(acc_ref)

    a = a_ref[...]
    b = b_ref[...]
    acc_ref[...] += jnp.dot(a.astype(jnp.float32), b.astype(jnp.float32),
                            preferred_element_type=jnp.float32)

    @pl.when(pl.program_id(2) == nk - 1)
    def _():
        o_ref[...] = acc_ref[...]


def _mm(a, b, bm=BLK, bn=BLK, bk=BLK):
    m, k = a.shape
    k2, n = b.shape
    bm = min(bm, m)
    bn = min(bn, n)
    bk = min(bk, k)
    grid = (m // bm, n // bn, k // bk)
    return pl.pallas_call(
        functools.partial(_mm_body, nk=grid[2]),
        out_shape=jax.ShapeDtypeStruct((m, n), jnp.float32),
        grid=grid,
        in_specs=[
            pl.BlockSpec((bm, bk), lambda i, j, h: (i, h)),
            pl.BlockSpec((bk, bn), lambda i, j, h: (h, j)),
        ],
        out_specs=pl.BlockSpec((bm, bn), lambda i, j, h: (i, j)),
        scratch_shapes=[pltpu.VMEM((bm, bn), jnp.float32)],
    )(a, b)


def _aug_body(a_ref, b_ref, o_ref, acc_ref, *, nk, blk):
    @pl.when(pl.program_id(2) == 0)
    def _():
        acc_ref[...] = jnp.zeros_like(acc_ref)

    acc_ref[...] += jnp.dot(a_ref[...], b_ref[...],
                            preferred_element_type=jnp.float32)

    @pl.when(pl.program_id(2) == nk - 1)
    def _():
        acc = acc_ref[...]
        i = pl.program_id(0)
        j = pl.program_id(1)

        @pl.when(i == j)
        def _():
            r = jax.lax.broadcasted_iota(jnp.int32, (blk, blk), 0)
            c = jax.lax.broadcasted_iota(jnp.int32, (blk, blk), 1)
            acc_ref[...] = jnp.where(r == c, 0.0, acc)

        o_ref[...] = acc_ref[...].astype(jnp.bfloat16)


def _augment(a_bf):
    n = a_bf.shape[0]
    blk = min(BLK, n)
    grid = (n // blk, n // blk, n // blk)
    return pl.pallas_call(
        functools.partial(_aug_body, nk=grid[2], blk=blk),
        out_shape=jax.ShapeDtypeStruct((n, n), jnp.bfloat16),
        grid=grid,
        in_specs=[
            pl.BlockSpec((blk, blk), lambda i, j, h: (i, h)),
            pl.BlockSpec((blk, blk), lambda i, j, h: (h, j)),
        ],
        out_specs=pl.BlockSpec((blk, blk), lambda i, j, h: (i, j)),
        scratch_shapes=[pltpu.VMEM((blk, blk), jnp.float32)],
    )(a_bf, a_bf)


def _prep_body(a_ref, at_ref, r_ref, c_ref, *, nk, blk):
    i = pl.program_id(0)
    k = pl.program_id(1)
    a = a_ref[...].astype(jnp.float32)

    @pl.when(k == 0)
    def _():
        r_ref[...] = jnp.zeros_like(r_ref)
        c_ref[...] = jnp.zeros_like(c_ref)

    r_ref[...] += jnp.sum(a, axis=1, keepdims=True) + jnp.zeros(
        (blk, 128), jnp.float32)

    rr = jax.lax.broadcasted_iota(jnp.int32, (blk, blk), 0)
    cc = jax.lax.broadcasted_iota(jnp.int32, (blk, blk), 1)
    diag_m = (rr == cc)

    @pl.when(i == k)
    def _():
        c_ref[...] += jnp.sum(jnp.where(diag_m, a, 0.0), axis=1,
                              keepdims=True) + jnp.zeros((blk, 128),
                                                         jnp.float32)
        at_ref[...] = jnp.where(diag_m, 1.0, a).astype(jnp.bfloat16)

    @pl.when(i != k)
    def _():
        at_ref[...] = a.astype(jnp.bfloat16)


def _prep(a):
    n = a.shape[0]
    blk = min(BLK, n)
    grid = (n // blk, n // blk)
    at, r, c = pl.pallas_call(
        functools.partial(_prep_body, nk=grid[1], blk=blk),
        out_shape=[
            jax.ShapeDtypeStruct((n, n), jnp.bfloat16),
            jax.ShapeDtypeStruct((n, 128), jnp.float32),
            jax.ShapeDtypeStruct((n, 128), jnp.float32),
        ],
        grid=grid,
        in_specs=[pl.BlockSpec((blk, blk), lambda i, k: (i, k))],
        out_specs=[
            pl.BlockSpec((blk, blk), lambda i, k: (i, k)),
            pl.BlockSpec((blk, 128), lambda i, k: (i, 0)),
            pl.BlockSpec((blk, 128), lambda i, k: (i, 0)),
        ],
    )(a)
    return at, r[:, 0], c[:, 0]


def _norm_vecs(r, c):
    extra = jnp.where(c == 0, 2.0, 0.0)
    deg = r + extra
    dinv = jnp.where(deg > 0, jax.lax.rsqrt(deg), 0.0)
    coeff = (c - 1.0 + extra) * dinv * dinv
    return dinv, coeff


def _gcn_conv(at_bf, dinv, coeff, x, W, b, relu, row_scale=None):
    if row_scale is not None:
        x = x * row_scale[:, None]
    z = _mm(x, W)
    zs = z * dinv[:, None]
    y = _mm(at_bf, zs) * dinv[:, None] + coeff[:, None] * z + b
    if relu:
        y = jax.nn.relu(y)
    return y


def kernel(x, edge_index, W_down0, b_down0, W_down1, b_down1, W_down2,
           b_down2, W_down3, b_down3, p_pool1, p_pool2, p_pool3,
           W_up0, b_up0, W_up1, b_up1, W_up2, b_up2):
    n = x.shape[0]
    A = jnp.zeros((n, n), jnp.float32).at[edge_index[1], edge_index[0]].add(
        jnp.ones((edge_index.shape[1],), jnp.float32))

    at, r, c = _prep(A)
    dinv, coeff = _norm_vecs(r, c)
    x = _gcn_conv(at, dinv, coeff, x, W_down0, b_down0, relu=True)

    xs = [x]
    ats = [at]
    norms = [(dinv, coeff)]
    perms = []
    Wd = [(W_down1, b_down1), (W_down2, b_down2), (W_down3, b_down3)]
    ps = [p_pool1, p_pool2, p_pool3]

    for i in range(DEPTH):
        A2 = _augment(at)
        p = ps[i]
        pn = p / jnp.linalg.norm(p)
        P = jnp.zeros((128, 128), jnp.float32).at[:, 0].set(pn)
        score = _mm(x, P)[:, 0]
        k = int(math.ceil(RATIO * x.shape[0]))
        vals, perm = jax.lax.top_k(score, k)
        scale = jnp.tanh(vals)
        Ap = A2[perm][:, perm]

        at, r, c = _prep(Ap)
        dinv, coeff = _norm_vecs(r, c)
        xg = x[perm]
        x = _gcn_conv(at, dinv, coeff, xg, Wd[i][0], Wd[i][1], relu=True,
                      row_scale=scale)
        if i < DEPTH - 1:
            xs.append(x)
            ats.append(at)
            norms.append((dinv, coeff))
        perms.append(perm)

    Wu = [(W_up0, b_up0), (W_up1, b_up1), (W_up2, b_up2)]
    for i in range(DEPTH):
        j = DEPTH - 1 - i
        res = xs[j]
        at = ats[j]
        dinv, coeff = norms[j]
        perm = perms[j]
        Wt, bt = Wu[i]
        h = _mm(res, Wt[:128]) + jnp.zeros(
            (res.shape[0], Wt.shape[1]), jnp.float32).at[perm].set(
                _mm(x, Wt[128:]))
        hs = h * dinv[:, None]
        y = _mm(at, hs) * dinv[:, None] + coeff[:, None] * h + bt
        if i < DEPTH - 1:
            y = jax.nn.relu(y)
        x = y
    return x


# speedup vs baseline: 1.4889x; 1.3085x over previous
import functools
import math

import jax
import jax.numpy as jnp
from jax.experimental import pallas as pl
from jax.experimental.pallas import tpu as pltpu

DEPTH = 3
RATIO = 0.5
BLK = 512


def _mm_body(a_ref, b_ref, o_ref, acc_ref, *, nk):
    @pl.when(pl.program_id(2) == 0)
    def _():
        acc_ref[...] = jnp.zeros_like(acc_ref)

    a = a_ref[...]
    b = b_ref[...]
    acc_ref[...] += jnp.dot(a.astype(jnp.float32), b.astype(jnp.float32),
                            preferred_element_type=jnp.float32)

    @pl.when(pl.program_id(2) == nk - 1)
    def _():
        o_ref[...] = acc_ref[...]


def _mm(a, b, bm=BLK, bn=BLK, bk=BLK):
    m, k = a.shape
    k2, n = b.shape
    bm = min(bm, m)
    bn = min(bn, n)
    bk = min(bk, k)
    grid = (m // bm, n // bn, k // bk)
    return pl.pallas_call(
        functools.partial(_mm_body, nk=grid[2]),
        out_shape=jax.ShapeDtypeStruct((m, n), jnp.float32),
        grid=grid,
        in_specs=[
            pl.BlockSpec((bm, bk), lambda i, j, h: (i, h)),
            pl.BlockSpec((bk, bn), lambda i, j, h: (h, j)),
        ],
        out_specs=pl.BlockSpec((bm, bn), lambda i, j, h: (i, j)),
        scratch_shapes=[pltpu.VMEM((bm, bn), jnp.float32)],
    )(a, b)


def _aug_body(a_ref, b_ref, o_ref, acc_ref, *, nk, blk):
    @pl.when(pl.program_id(2) == 0)
    def _():
        acc_ref[...] = jnp.zeros_like(acc_ref)

    acc_ref[...] += jnp.dot(a_ref[...], b_ref[...],
                            preferred_element_type=jnp.float32)

    @pl.when(pl.program_id(2) == nk - 1)
    def _():
        acc = acc_ref[...]
        i = pl.program_id(0)
        j = pl.program_id(1)

        @pl.when(i == j)
        def _():
            r = jax.lax.broadcasted_iota(jnp.int32, (blk, blk), 0)
            c = jax.lax.broadcasted_iota(jnp.int32, (blk, blk), 1)
            acc_ref[...] = jnp.where(r == c, 0.0, acc)

        o_ref[...] = acc_ref[...].astype(jnp.bfloat16)


def _augment(a_bf):
    n = a_bf.shape[0]
    blk = min(1024, n)
    bk = min(BLK, n)
    grid = (n // blk, n // blk, n // bk)
    return pl.pallas_call(
        functools.partial(_aug_body, nk=grid[2], blk=blk),
        out_shape=jax.ShapeDtypeStruct((n, n), jnp.bfloat16),
        grid=grid,
        in_specs=[
            pl.BlockSpec((blk, bk), lambda i, j, h: (i, h)),
            pl.BlockSpec((bk, blk), lambda i, j, h: (h, j)),
        ],
        out_specs=pl.BlockSpec((blk, blk), lambda i, j, h: (i, j)),
        scratch_shapes=[pltpu.VMEM((blk, blk), jnp.float32)],
    )(a_bf, a_bf)


def _prep_body(a_ref, at_ref, r_ref, c_ref, *, nk, blk):
    i = pl.program_id(0)
    k = pl.program_id(1)
    a = a_ref[...].astype(jnp.float32)

    @pl.when(k == 0)
    def _():
        r_ref[...] = jnp.zeros_like(r_ref)
        c_ref[...] = jnp.zeros_like(c_ref)

    r_ref[...] += jnp.sum(a, axis=1, keepdims=True) + jnp.zeros(
        (blk, 128), jnp.float32)

    rr = jax.lax.broadcasted_iota(jnp.int32, (blk, blk), 0)
    cc = jax.lax.broadcasted_iota(jnp.int32, (blk, blk), 1)
    diag_m = (rr == cc)

    @pl.when(i == k)
    def _():
        c_ref[...] += jnp.sum(jnp.where(diag_m, a, 0.0), axis=1,
                              keepdims=True) + jnp.zeros((blk, 128),
                                                         jnp.float32)
        at_ref[...] = jnp.where(diag_m, 1.0, a).astype(jnp.bfloat16)

    @pl.when(i != k)
    def _():
        at_ref[...] = a.astype(jnp.bfloat16)


def _prep(a):
    n = a.shape[0]
    blk = min(BLK, n)
    grid = (n // blk, n // blk)
    at, r, c = pl.pallas_call(
        functools.partial(_prep_body, nk=grid[1], blk=blk),
        out_shape=[
            jax.ShapeDtypeStruct((n, n), jnp.bfloat16),
            jax.ShapeDtypeStruct((n, 128), jnp.float32),
            jax.ShapeDtypeStruct((n, 128), jnp.float32),
        ],
        grid=grid,
        in_specs=[pl.BlockSpec((blk, blk), lambda i, k: (i, k))],
        out_specs=[
            pl.BlockSpec((blk, blk), lambda i, k: (i, k)),
            pl.BlockSpec((blk, 128), lambda i, k: (i, 0)),
            pl.BlockSpec((blk, 128), lambda i, k: (i, 0)),
        ],
    )(a)
    return at, r[:, 0], c[:, 0]


def _norm_vecs(r, c):
    extra = jnp.where(c == 0, 2.0, 0.0)
    deg = r + extra
    dinv = jnp.where(deg > 0, jax.lax.rsqrt(deg), 0.0)
    coeff = (c - 1.0 + extra) * dinv * dinv
    return dinv, coeff


def _gcn_conv(at_bf, dinv, coeff, x, W, b, relu, row_scale=None):
    if row_scale is not None:
        x = x * row_scale[:, None]
    z = _mm(x, W)
    zs = z * dinv[:, None]
    y = _mm(at_bf, zs) * dinv[:, None] + coeff[:, None] * z + b
    if relu:
        y = jax.nn.relu(y)
    return y


def kernel(x, edge_index, W_down0, b_down0, W_down1, b_down1, W_down2,
           b_down2, W_down3, b_down3, p_pool1, p_pool2, p_pool3,
           W_up0, b_up0, W_up1, b_up1, W_up2, b_up2):
    n = x.shape[0]
    A = jnp.zeros((n, n), jnp.float32).at[edge_index[1], edge_index[0]].add(
        jnp.ones((edge_index.shape[1],), jnp.float32))

    at, r, c = _prep(A)
    dinv, coeff = _norm_vecs(r, c)
    x = _gcn_conv(at, dinv, coeff, x, W_down0, b_down0, relu=True)

    xs = [x]
    ats = [at]
    norms = [(dinv, coeff)]
    perms = []
    Wd = [(W_down1, b_down1), (W_down2, b_down2), (W_down3, b_down3)]
    ps = [p_pool1, p_pool2, p_pool3]

    for i in range(DEPTH):
        A2 = _augment(at)
        p = ps[i]
        pn = p / jnp.linalg.norm(p)
        P = jnp.zeros((128, 128), jnp.float32).at[:, 0].set(pn)
        score = _mm(x, P)[:, 0]
        k = int(math.ceil(RATIO * x.shape[0]))
        vals, perm = jax.lax.top_k(score, k)
        scale = jnp.tanh(vals)
        Ap = A2[perm][:, perm]

        at, r, c = _prep(Ap)
        dinv, coeff = _norm_vecs(r, c)
        xg = x[perm]
        x = _gcn_conv(at, dinv, coeff, xg, Wd[i][0], Wd[i][1], relu=True,
                      row_scale=scale)
        if i < DEPTH - 1:
            xs.append(x)
            ats.append(at)
            norms.append((dinv, coeff))
        perms.append(perm)

    Wu = [(W_up0, b_up0), (W_up1, b_up1), (W_up2, b_up2)]
    for i in range(DEPTH):
        j = DEPTH - 1 - i
        res = xs[j]
        at = ats[j]
        dinv, coeff = norms[j]
        perm = perms[j]
        Wt, bt = Wu[i]
        h = _mm(res, Wt[:128]) + jnp.zeros(
            (res.shape[0], Wt.shape[1]), jnp.float32).at[perm].set(
                _mm(x, Wt[128:]))
        hs = h * dinv[:, None]
        y = _mm(at, hs) * dinv[:, None] + coeff[:, None] * h + bt
        if i < DEPTH - 1:
            y = jax.nn.relu(y)
        x = y
    return x
